# baseline (speedup 1.0000x reference)
import functools

import jax
import jax.numpy as jnp
from jax import lax
from jax.experimental import pallas as pl
from jax.experimental.pallas import tpu as pltpu
from jax.experimental.pallas import tpu_sc as plsc

NC = 2
NS = 16
NW = NC * NS

H_GROUPS = 2
B_GROUPS = 16
CH = 512


def _make_kernel(HIST, BATCH, V, D):
    assert NW == H_GROUPS * B_GROUPS
    h_per_w = HIST // H_GROUPS
    b_per_w = BATCH // B_GROUPS
    assert h_per_w * H_GROUPS == HIST and b_per_w * B_GROUPS == BATCH
    cpb = b_per_w // CH
    n_chunks = h_per_w * cpb
    assert cpb * CH == b_per_w and n_chunks % 2 == 0 and n_chunks >= 6

    mesh = plsc.VectorSubcoreMesh(core_axis_name="c", subcore_axis_name="s")

    @functools.partial(
        pl.kernel,
        out_type=jax.ShapeDtypeStruct((HIST, D, BATCH), jnp.float32),
        mesh=mesh,
        scratch_types=[
            pltpu.VMEM((CH,), jnp.int32),
            pltpu.VMEM((CH,), jnp.int32),
            pltpu.VMEM((CH, D), jnp.float32),
            pltpu.VMEM((CH, D), jnp.float32),
            pltpu.VMEM((D, CH), jnp.float32),
            pltpu.VMEM((D, CH), jnp.float32),
            pltpu.SemaphoreType.DMA,
            pltpu.SemaphoreType.DMA,
            pltpu.SemaphoreType.DMA,
            pltpu.SemaphoreType.DMA,
            pltpu.SemaphoreType.DMA,
            pltpu.SemaphoreType.DMA,
        ],
        compiler_params=pltpu.CompilerParams(
            use_tc_tiling_on_sc=False, needs_layout_passes=False),
    )
    def gather_kernel(x_hbm, w_hbm, out_hbm, idx0, idx1, rows0, rows1,
                      tb0, tb1, s_i0, s_i1, s_g0, s_g1, s_w0, s_w1):
        idx_v = [idx0, idx1]
        rows_v = [rows0, rows1]
        tbuf = [tb0, tb1]
        sem_i = [s_i0, s_i1]
        sem_g = [s_g0, s_g1]
        sem_w = [s_w0, s_w1]
        wid = lax.axis_index("s") * NC + lax.axis_index("c")
        hg = wid // B_GROUPS
        bg = wid % B_GROUPS
        h0 = hg * h_per_w
        b0 = bg * b_per_w

        def chunk_hb(c):
            h = h0 + c // cpb
            bb = b0 + (c % cpb) * CH
            return h, bb

        def idx_copy(c, b):
            h, bb = chunk_hb(c)
            return pltpu.make_async_copy(
                x_hbm.at[pl.ds(h * BATCH + bb, CH)], idx_v[b], sem_i[b])

        def gather(b):
            return pltpu.make_async_copy(
                w_hbm.at[idx_v[b]], rows_v[b], sem_g[b])

        def writeout(c, b):
            h, bb = chunk_hb(c)
            return pltpu.make_async_copy(
                tbuf[b], out_hbm.at[h, :, pl.ds(bb, CH)], sem_w[b])

        lane = lax.iota(jnp.int32, 16)
        cols = [jnp.full((16,), d, jnp.int32) for d in range(D)]

        def transpose(b):
            rows = rows_v[b]
            tb = tbuf[b]

            @plsc.parallel_loop(0, CH, 16, unroll=4)
            def _(j0):
                row_idx = j0 + lane
                for d in range(D):
                    tb[d, pl.ds(j0, 16)] = plsc.load_gather(
                        rows, [row_idx, cols[d]])

        for b in range(2):
            idx_copy(b, b).start()

        def outer(cc, _):
            for b in range(2):
                c = cc * 2 + b
                pb = 1 - b

                idx_copy(c, b).wait()
                gather(b).start()

                @pl.when(c >= 1)
                def _():
                    gather(pb).wait()

                @pl.when(jnp.logical_and(c >= 1, c + 1 < n_chunks))
                def _():
                    idx_copy(c + 1, pb).start()

                @pl.when(c >= 3)
                def _():
                    writeout(c - 3, pb).wait()

                @pl.when(c >= 1)
                def _():
                    transpose(pb)
                    writeout(c - 1, pb).start()
            return 0

        lax.fori_loop(0, n_chunks // 2, outer, 0)

        bl = (n_chunks - 1) % 2
        gather(bl).wait()
        writeout(n_chunks - 3, bl).wait()
        transpose(bl)
        writeout(n_chunks - 1, bl).start()
        writeout(n_chunks - 2, 1 - bl).wait()
        writeout(n_chunks - 1, bl).wait()

    return gather_kernel


def kernel(x, W):
    B_, H = x.shape
    V, D = W.shape
    flat = jnp.transpose(x).reshape(-1).astype(jnp.int32)
    out = _make_kernel(H, B_, V, D)(flat, W)
    return jnp.transpose(out, (2, 0, 1))


# speedup vs baseline: 2.0242x; 2.0242x over previous
import functools

import jax
import jax.numpy as jnp
from jax import lax
from jax.experimental import pallas as pl
from jax.experimental.pallas import tpu as pltpu
from jax.experimental.pallas import tpu_sc as plsc

NC = 2
NS = 16
NW = NC * NS

H_GROUPS = 2
B_GROUPS = 16
CH = 512


def _make_kernel(HIST, BATCH, V, D):
    assert NW == H_GROUPS * B_GROUPS
    h_per_w = HIST // H_GROUPS
    b_per_w = BATCH // B_GROUPS
    assert h_per_w * H_GROUPS == HIST and b_per_w * B_GROUPS == BATCH
    cpb = b_per_w // CH
    n_chunks = h_per_w * cpb
    assert cpb * CH == b_per_w and n_chunks % 2 == 0 and n_chunks >= 6

    mesh = plsc.VectorSubcoreMesh(core_axis_name="c", subcore_axis_name="s")

    @functools.partial(
        pl.kernel,
        out_type=jax.ShapeDtypeStruct((HIST, D, BATCH), jnp.float32),
        mesh=mesh,
        scratch_types=[
            pltpu.VMEM((CH,), jnp.int32),
            pltpu.VMEM((CH,), jnp.int32),
            pltpu.VMEM((CH, D), jnp.float32),
            pltpu.VMEM((CH, D), jnp.float32),
            pltpu.VMEM((D, CH), jnp.float32),
            pltpu.VMEM((D, CH), jnp.float32),
            pltpu.SemaphoreType.DMA,
            pltpu.SemaphoreType.DMA,
            pltpu.SemaphoreType.DMA,
            pltpu.SemaphoreType.DMA,
            pltpu.SemaphoreType.DMA,
            pltpu.SemaphoreType.DMA,
        ],
        compiler_params=pltpu.CompilerParams(
            use_tc_tiling_on_sc=False, needs_layout_passes=False),
    )
    def gather_kernel(x_hbm, w_hbm, out_hbm, idx0, idx1, rows0, rows1,
                      tb0, tb1, s_i0, s_i1, s_g0, s_g1, s_w0, s_w1):
        idx_v = [idx0, idx1]
        rows_v = [rows0, rows1]
        tbuf = [tb0, tb1]
        sem_i = [s_i0, s_i1]
        sem_g = [s_g0, s_g1]
        sem_w = [s_w0, s_w1]
        wid = lax.axis_index("s") * NC + lax.axis_index("c")
        hg = wid // B_GROUPS
        bg = wid % B_GROUPS
        h0 = hg * h_per_w
        b0 = bg * b_per_w

        def chunk_hb(c):
            h = h0 + c // cpb
            bb = b0 + (c % cpb) * CH
            return h, bb

        def idx_copy(c, b):
            h, bb = chunk_hb(c)
            return pltpu.make_async_copy(
                x_hbm.at[pl.ds(h * BATCH + bb, CH)], idx_v[b], sem_i[b])

        def gather(b):
            return pltpu.make_async_copy(
                w_hbm.at[idx_v[b]], rows_v[b], sem_g[b])

        def writeout(c, b):
            h, bb = chunk_hb(c)
            return pltpu.make_async_copy(
                tbuf[b], out_hbm.at[h, :, pl.ds(bb, CH)], sem_w[b])

        lane = lax.iota(jnp.int32, 16)
        diag = [(lane + s) % 16 + d0 for d0 in range(0, D, 16) for s in range(16)]

        def transpose(b):
            rows = rows_v[b]
            tb = tbuf[b]

            @plsc.parallel_loop(0, CH, 16, unroll=2)
            def _(j0):
                rowv = j0 + lane
                for colv in diag:
                    plsc.store_scatter(
                        tb, [colv, rowv], plsc.load_gather(rows, [rowv, colv]))

        for b in range(2):
            idx_copy(b, b).start()

        def outer(cc, _):
            for b in range(2):
                c = cc * 2 + b
                pb = 1 - b

                idx_copy(c, b).wait()
                gather(b).start()

                @pl.when(c >= 1)
                def _():
                    gather(pb).wait()

                @pl.when(jnp.logical_and(c >= 1, c + 1 < n_chunks))
                def _():
                    idx_copy(c + 1, pb).start()

                @pl.when(c >= 3)
                def _():
                    writeout(c - 3, pb).wait()

                @pl.when(c >= 1)
                def _():
                    transpose(pb)
                    writeout(c - 1, pb).start()
            return 0

        lax.fori_loop(0, n_chunks // 2, outer, 0)

        bl = (n_chunks - 1) % 2
        gather(bl).wait()
        writeout(n_chunks - 3, bl).wait()
        transpose(bl)
        writeout(n_chunks - 1, bl).start()
        writeout(n_chunks - 2, 1 - bl).wait()
        writeout(n_chunks - 1, bl).wait()

    return gather_kernel


def kernel(x, W):
    B_, H = x.shape
    V, D = W.shape
    flat = jnp.transpose(x).reshape(-1).astype(jnp.int32)
    out = _make_kernel(H, B_, V, D)(flat, W)
    return jnp.transpose(out, (2, 0, 1))


# speedup vs baseline: 2.8378x; 1.4020x over previous
import functools

import jax
import jax.numpy as jnp
from jax import lax
from jax.experimental import pallas as pl
from jax.experimental.pallas import tpu as pltpu
from jax.experimental.pallas import tpu_sc as plsc

NC = 2
NS = 16
NW = NC * NS

H_GROUPS = 2
B_GROUPS = 16
CH = 512


def _make_kernel(HIST, BATCH, V, D):
    assert NW == H_GROUPS * B_GROUPS
    assert D == 32 and BATCH % (B_GROUPS * CH) == 0 and CH % 128 == 0
    h_per_w = HIST // H_GROUPS
    b_per_w = BATCH // B_GROUPS
    assert h_per_w * H_GROUPS == HIST
    cpb = b_per_w // CH
    n_chunks = h_per_w * cpb
    assert n_chunks % 2 == 0 and n_chunks >= 6
    TRS = D // 8
    STRIP = (CH // 128) * 1024

    mesh = plsc.VectorSubcoreMesh(core_axis_name="c", subcore_axis_name="s")

    @functools.partial(
        pl.kernel,
        out_type=jax.ShapeDtypeStruct((HIST, TRS, (BATCH // 128) * 1024),
                                      jnp.float32),
        mesh=mesh,
        scratch_types=[
            pltpu.VMEM((CH,), jnp.int32),
            pltpu.VMEM((CH,), jnp.int32),
            pltpu.VMEM((CH, D), jnp.float32),
            pltpu.VMEM((CH, D), jnp.float32),
            pltpu.VMEM((TRS * STRIP,), jnp.float32),
            pltpu.VMEM((TRS * STRIP,), jnp.float32),
            pltpu.SemaphoreType.DMA,
            pltpu.SemaphoreType.DMA,
            pltpu.SemaphoreType.DMA,
            pltpu.SemaphoreType.DMA,
            pltpu.SemaphoreType.DMA,
            pltpu.SemaphoreType.DMA,
        ],
        compiler_params=pltpu.CompilerParams(
            use_tc_tiling_on_sc=False, needs_layout_passes=False),
    )
    def gather_kernel(x_hbm, w_hbm, out_hbm, idx0, idx1, rows0, rows1,
                      tb0, tb1, s_i0, s_i1, s_g0, s_g1, s_w0, s_w1):
        idx_v = [idx0, idx1]
        rows_v = [rows0, rows1]
        tbuf = [tb0, tb1]
        sem_i = [s_i0, s_i1]
        sem_g = [s_g0, s_g1]
        sem_w = [s_w0, s_w1]
        wid = lax.axis_index("s") * NC + lax.axis_index("c")
        hg = wid // B_GROUPS
        bg = wid % B_GROUPS
        h0 = hg * h_per_w
        b0 = bg * b_per_w

        def chunk_hb(c):
            h = h0 + c // cpb
            bb = b0 + (c % cpb) * CH
            return h, bb

        def idx_copy(c, b):
            h, bb = chunk_hb(c)
            return pltpu.make_async_copy(
                x_hbm.at[pl.ds(h * BATCH + bb, CH)], idx_v[b], sem_i[b])

        def gather(b):
            return pltpu.make_async_copy(
                w_hbm.at[idx_v[b]], rows_v[b], sem_g[b])

        def write_descs(c, b):
            h, bb = chunk_hb(c)
            inner0 = (bb // 128) * 1024
            return [
                pltpu.make_async_copy(
                    tbuf[b].at[pl.ds(tr * STRIP, STRIP)],
                    out_hbm.at[h, tr, pl.ds(inner0, STRIP)],
                    sem_w[b])
                for tr in range(TRS)
            ]

        lane = lax.iota(jnp.int32, 16)
        diag = [(lane + s) % 16 + d0 for d0 in range(0, D, 16) for s in range(16)]
        dbase = [(dv // 8) * STRIP + (dv % 8) * 128 + lane for dv in diag]

        def transpose(b):
            rows = rows_v[b]
            tb = tbuf[b]

            @plsc.parallel_loop(0, CH, 16, unroll=2)
            def _(j0):
                rowv = j0 + lane
                scal = (j0 // 128) * 1024 + j0 % 128
                for colv, basev in zip(diag, dbase):
                    plsc.store_scatter(
                        tb, [basev + scal],
                        plsc.load_gather(rows, [rowv, colv]))

        for b in range(2):
            idx_copy(b, b).start()

        def outer(cc, _):
            for b in range(2):
                c = cc * 2 + b
                pb = 1 - b

                idx_copy(c, b).wait()
                gather(b).start()

                @pl.when(c >= 1)
                def _():
                    gather(pb).wait()

                @pl.when(jnp.logical_and(c >= 1, c + 1 < n_chunks))
                def _():
                    idx_copy(c + 1, pb).start()

                @pl.when(c >= 3)
                def _():
                    for d_ in write_descs(c - 3, pb):
                        d_.wait()

                @pl.when(c >= 1)
                def _():
                    transpose(pb)
                    for d_ in write_descs(c - 1, pb):
                        d_.start()
            return 0

        lax.fori_loop(0, n_chunks // 2, outer, 0)

        bl = (n_chunks - 1) % 2
        gather(bl).wait()
        for d_ in write_descs(n_chunks - 3, bl):
            d_.wait()
        transpose(bl)
        for d_ in write_descs(n_chunks - 1, bl):
            d_.start()
        for d_ in write_descs(n_chunks - 2, 1 - bl):
            d_.wait()
        for d_ in write_descs(n_chunks - 1, bl):
            d_.wait()

    return gather_kernel


def kernel(x, W):
    B_, H = x.shape
    V, D = W.shape
    flat = jnp.transpose(x).reshape(-1).astype(jnp.int32)
    out = _make_kernel(H, B_, V, D)(flat, W)
    out = out.reshape(H, D // 8, B_ // 128, 8, 128)
    out = jnp.transpose(out, (2, 4, 0, 1, 3))
    return out.reshape(B_, H, D)
